# baseline (speedup 1.0000x reference)
import functools

import jax
import jax.numpy as jnp
from jax import lax
from jax.experimental import pallas as pl
from jax.experimental.pallas import tpu as pltpu
from jax.experimental.pallas import tpu_sc as plsc

F = 128
NRBF = 20
NRBF_PAD = 24
CUTOFF = 5.0

NC = 2
NS = 16
NW = NC * NS
LANES = 16

_MESH = plsc.VectorSubcoreMesh(core_axis_name="c", subcore_axis_name="s")


def _flags_body(nbr_ref, gt_ref, lt_ref):
    blk = nbr_ref[...]
    gt = jnp.any(blk[:, 0] > blk[:, 1]).astype(jnp.float32)
    lt = jnp.any(blk[:, 1] > blk[:, 0]).astype(jnp.float32)

    @pl.when(pl.program_id(0) == 0)
    def _():
        gt_ref[...] = jnp.zeros_like(gt_ref)
        lt_ref[...] = jnp.zeros_like(lt_ref)

    gt_ref[...] = jnp.maximum(gt_ref[...], gt)
    lt_ref[...] = jnp.maximum(lt_ref[...], lt)


def _directed_flags(nbr_list):
    n = nbr_list.shape[0]
    rows = 8000
    grid = n // rows
    return pl.pallas_call(
        _flags_body,
        grid=(grid,),
        in_specs=[pl.BlockSpec((rows, 2), lambda i: (i, 0))],
        out_specs=[
            pl.BlockSpec((1, 128), lambda i: (0, 0)),
            pl.BlockSpec((1, 128), lambda i: (0, 0)),
        ],
        out_shape=[
            jax.ShapeDtypeStruct((1, 128), jnp.float32),
            jax.ShapeDtypeStruct((1, 128), jnp.float32),
        ],
        name="directed_flags",
    )(nbr_list)


def _phi_body(s_ref, p0_ref, p1_ref, w1_ref, b1_ref, w2_ref, b2_ref,
              snew_ref, a_ref, b_ref, c_ref):
    s = s_ref[...] + p0_ref[...] + p1_ref[...]
    snew_ref[...] = s
    h = jnp.dot(s, w1_ref[...], preferred_element_type=jnp.float32) + b1_ref[...]
    h = h * jax.nn.sigmoid(h)
    phi = jnp.dot(h, w2_ref[...], preferred_element_type=jnp.float32) + b2_ref[...]
    a_ref[...] = phi[:, 0:128]
    b_ref[...] = phi[:, 128:256]
    c_ref[...] = phi[:, 256:384]


def _phi_panels(s, p0, p1, W1, b1, W2, b2):
    n = s.shape[0]
    rows = 1000
    grid = n // rows
    return pl.pallas_call(
        _phi_body,
        grid=(grid,),
        in_specs=[
            pl.BlockSpec((rows, F), lambda i: (i, 0)),
            pl.BlockSpec((rows, F), lambda i: (i, 0)),
            pl.BlockSpec((rows, F), lambda i: (i, 0)),
            pl.BlockSpec((F, F), lambda i: (0, 0)),
            pl.BlockSpec((F,), lambda i: (0,)),
            pl.BlockSpec((F, 3 * F), lambda i: (0, 0)),
            pl.BlockSpec((3 * F,), lambda i: (0,)),
        ],
        out_specs=[
            pl.BlockSpec((rows, F), lambda i: (i, 0)),
            pl.BlockSpec((rows, F), lambda i: (i, 0)),
            pl.BlockSpec((rows, F), lambda i: (i, 0)),
            pl.BlockSpec((rows, F), lambda i: (i, 0)),
        ],
        out_shape=[
            jax.ShapeDtypeStruct((n, F), jnp.float32),
            jax.ShapeDtypeStruct((n, F), jnp.float32),
            jax.ShapeDtypeStruct((n, F), jnp.float32),
            jax.ShapeDtypeStruct((n, F), jnp.float32),
        ],
        name="phi_panels",
    )(s, p0, p1, W1, b1, W2, b2)


def _d_body(rbf_ref, envm_ref, u_ref, wd_ref, bd_ref,
            da_ref, db_ref, dcx_ref, dcy_ref, dcz_ref):
    d = jnp.dot(rbf_ref[...], wd_ref[...], preferred_element_type=jnp.float32)
    d = d + envm_ref[...] * bd_ref[...]
    da_ref[...] = d[:, 0:128]
    db_ref[...] = d[:, 128:256]
    dc = d[:, 256:384]
    dcx_ref[...] = dc * u_ref[:, 0:1]
    dcy_ref[...] = dc * u_ref[:, 1:2]
    dcz_ref[...] = dc * u_ref[:, 2:3]


def _d_panels(rbfenv, envm_col, u16, Wd_pad, bd):
    e = rbfenv.shape[0]
    rows = 512
    grid = e // rows
    return pl.pallas_call(
        _d_body,
        grid=(grid,),
        in_specs=[
            pl.BlockSpec((rows, NRBF_PAD), lambda i: (i, 0)),
            pl.BlockSpec((rows, 1), lambda i: (i, 0)),
            pl.BlockSpec((rows, 16), lambda i: (i, 0)),
            pl.BlockSpec((NRBF_PAD, 3 * F), lambda i: (0, 0)),
            pl.BlockSpec((1, 3 * F), lambda i: (0, 0)),
        ],
        out_specs=[pl.BlockSpec((rows, F), lambda i: (i, 0))] * 5,
        out_shape=[jax.ShapeDtypeStruct((e, F), jnp.float32)] * 5,
        name="d_panels",
    )(rbfenv, envm_col, u16, Wd_pad, bd)


def _merge3_body(a_ref, b_ref, c_ref, o_ref):
    o_ref[...] = a_ref[...] + b_ref[...] + c_ref[...]


def _merge3(a, b, c):
    n, k = a.shape
    rows = 1000
    grid = n // rows
    return pl.pallas_call(
        _merge3_body,
        grid=(grid,),
        in_specs=[pl.BlockSpec((rows, k), lambda i: (i, 0))] * 3,
        out_specs=pl.BlockSpec((rows, k), lambda i: (i, 0)),
        out_shape=jax.ShapeDtypeStruct((n, k), jnp.float32),
        name="merge3",
    )(a, b, c)


def _acc_zero(z_hbm, acc, s, rpt, extra_off, extra):
    r0 = s * rpt
    pltpu.sync_copy(z_hbm.at[pl.ds(r0, rpt), :], acc.at[pl.ds(r0, rpt), :])
    if extra:
        @pl.when(s == NS - 1)
        def _():
            pltpu.sync_copy(z_hbm.at[pl.ds(extra_off, extra), :],
                            acc.at[pl.ds(extra_off, extra), :])


def _acc_dump(acc, out_hbm, c, s, n, rpt, extra_off, extra):
    r0 = s * rpt
    pltpu.sync_copy(acc.at[pl.ds(r0, rpt), :],
                    out_hbm.at[pl.ds(c * n + r0, rpt), :])
    if extra:
        @pl.when(s == NS - 1)
        def _():
            pltpu.sync_copy(acc.at[pl.ds(extra_off, extra), :],
                            out_hbm.at[pl.ds(c * n + extra_off, extra), :])


def _row_split(n):
    rpt = (n // NS) // 8 * 8
    extra_off = rpt * NS
    extra = n - extra_off
    return rpt, extra_off, extra


def _xyz_gather(xyzpad, nbr_i, nbr_j):
    e2 = nbr_i.shape[0]
    epw = e2 // NW
    eb = 160
    nblk = epw // eb

    @functools.partial(
        pl.kernel,
        out_type=jax.ShapeDtypeStruct((e2, 16), jnp.float32),
        mesh=_MESH,
        scratch_types=[
            pltpu.VMEM((eb,), jnp.int32),
            pltpu.VMEM((eb,), jnp.int32),
            pltpu.VMEM((eb, 128), jnp.float32),
            pltpu.VMEM((eb, 128), jnp.float32),
            pltpu.VMEM((eb, 16), jnp.float32),
        ],
        name="sc_xyz_gather",
    )
    def k(xyz_hbm, ni_hbm, nj_hbm, r_hbm, ii_v, jj_v, gi_v, gj_v, o_v):
        c = lax.axis_index("c")
        s = lax.axis_index("s")
        wid = c * NS + s

        def body(b, _):
            off = wid * epw + b * eb
            pltpu.sync_copy(ni_hbm.at[pl.ds(off, eb)], ii_v)
            pltpu.sync_copy(nj_hbm.at[pl.ds(off, eb)], jj_v)
            pltpu.sync_copy(xyz_hbm.at[ii_v], gi_v)
            pltpu.sync_copy(xyz_hbm.at[jj_v], gj_v)

            def edge(ei, _):
                o_v[ei, :] = gj_v[ei, 0:16] - gi_v[ei, 0:16]
                return 0

            lax.fori_loop(0, eb, edge, 0)
            pltpu.sync_copy(o_v, r_hbm.at[pl.ds(off, eb), :])
            return 0

        lax.fori_loop(0, nblk, body, 0)

    return k(xyzpad, nbr_i, nbr_j)


def _sc_pass_s(nbr_i, nbr_j, phiB, DB, zeros128):
    n = phiB.shape[0]
    e2 = nbr_i.shape[0]
    epw = e2 // NW
    eb = 80
    nsup = 25
    nblk = epw // eb
    ngrp = nblk // nsup
    rpt, extra_off, extra = _row_split(n)

    @functools.partial(
        pl.kernel,
        out_type=jax.ShapeDtypeStruct((2 * n, F), jnp.float32),
        mesh=_MESH,
        scratch_types=[
            pltpu.VMEM_SHARED((n, F), jnp.float32),
            pltpu.VMEM((nsup * eb,), jnp.int32),
            pltpu.VMEM((eb,), jnp.int32),
            pltpu.VMEM((eb, F), jnp.float32),
            pltpu.VMEM((eb, F), jnp.float32),
            pltpu.SemaphoreType.DMA,
        ],
        name="sc_pass_s",
    )
    def k(ni_hbm, nj_hbm, phib_hbm, db_hbm, z_hbm, out_hbm,
          acc, jj_v, ii_v, gb_v, db_v, sem):
        c = lax.axis_index("c")
        s = lax.axis_index("s")
        wid = c * NS + s
        _acc_zero(z_hbm, acc, s, rpt, extra_off, extra)
        plsc.subcore_barrier()

        def grp(g, _):
            soff = wid * epw + g * nsup * eb
            pltpu.sync_copy(nj_hbm.at[pl.ds(soff, nsup * eb)], jj_v)

            def body(b, _):
                off = soff + b * eb
                ws = [
                    pltpu.async_copy(ni_hbm.at[pl.ds(off, eb)], ii_v, sem),
                    pltpu.async_copy(db_hbm.at[pl.ds(off, eb), :], db_v, sem),
                    pltpu.async_copy(
                        phib_hbm.at[jj_v.at[pl.ds(b * eb, eb)]], gb_v, sem),
                ]
                for w in ws:
                    w.wait()

                def edge(ei, _):
                    for kk in range(F // LANES):
                        sl = pl.ds(kk * LANES, LANES)
                        db_v[ei, sl] = db_v[ei, sl] * gb_v[ei, sl]
                    return 0

                lax.fori_loop(0, eb, edge, 0, unroll=4)
                pltpu.sync_copy(db_v, acc.at[ii_v], add=True)
                return 0

            lax.fori_loop(0, nsup, body, 0)
            return 0

        lax.fori_loop(0, ngrp, grp, 0)
        plsc.subcore_barrier()
        _acc_dump(acc, out_hbm, c, s, n, rpt, extra_off, extra)

    return k(nbr_i, nbr_j, phiB, DB, zeros128)


def _sc_pass_v(axis, nbr_i, nbr_j, phiA, phiC, DA, DCU, va, zeros128):
    n = phiA.shape[0]
    e2 = nbr_i.shape[0]
    epw = e2 // NW
    eb = 40
    nsup = 25
    nblk = epw // eb
    ngrp = nblk // nsup
    rpt, extra_off, extra = _row_split(n)
    has_v = va is not None

    scratch = [
        pltpu.VMEM_SHARED((n, F), jnp.float32),
        pltpu.VMEM((nsup * eb,), jnp.int32),
        pltpu.VMEM((eb,), jnp.int32),
        pltpu.VMEM((eb, F), jnp.float32),
        pltpu.VMEM((eb, F), jnp.float32),
        pltpu.SemaphoreType.DMA,
    ]
    if has_v:
        scratch += [
            pltpu.VMEM((eb, F), jnp.float32),
            pltpu.VMEM((eb, F), jnp.float32),
            pltpu.VMEM((eb, F), jnp.float32),
        ]

    def body_fn(*refs):
        if has_v:
            (ni_hbm, nj_hbm, pa_hbm, pc_hbm, da_hbm, dcu_hbm, va_hbm,
             z_hbm, out_hbm,
             acc, jj_v, ii_v, dc_v, gc_v, sem, da_v, ga_v, gv_v) = refs
        else:
            (ni_hbm, nj_hbm, pa_hbm, pc_hbm, da_hbm, dcu_hbm,
             z_hbm, out_hbm,
             acc, jj_v, ii_v, dc_v, gc_v, sem) = refs
            va_hbm = da_v = ga_v = gv_v = None
        c = lax.axis_index("c")
        s = lax.axis_index("s")
        wid = c * NS + s
        _acc_zero(z_hbm, acc, s, rpt, extra_off, extra)
        plsc.subcore_barrier()

        def grp(g, _):
            soff = wid * epw + g * nsup * eb
            pltpu.sync_copy(nj_hbm.at[pl.ds(soff, nsup * eb)], jj_v)

            def body(b, _):
                off = soff + b * eb
                jslice = jj_v.at[pl.ds(b * eb, eb)]
                ws = [
                    pltpu.async_copy(ni_hbm.at[pl.ds(off, eb)], ii_v, sem),
                    pltpu.async_copy(dcu_hbm.at[pl.ds(off, eb), :], dc_v, sem),
                    pltpu.async_copy(pc_hbm.at[jslice], gc_v, sem),
                ]
                if has_v:
                    ws += [
                        pltpu.async_copy(da_hbm.at[pl.ds(off, eb), :], da_v, sem),
                        pltpu.async_copy(pa_hbm.at[jslice], ga_v, sem),
                        pltpu.async_copy(va_hbm.at[jslice], gv_v, sem),
                    ]
                for w in ws:
                    w.wait()

                def edge(ei, _):
                    for kk in range(F // LANES):
                        sl = pl.ds(kk * LANES, LANES)
                        val = dc_v[ei, sl] * gc_v[ei, sl]
                        if has_v:
                            val = val + da_v[ei, sl] * ga_v[ei, sl] * gv_v[ei, sl]
                        dc_v[ei, sl] = val
                    return 0

                lax.fori_loop(0, eb, edge, 0, unroll=4)
                pltpu.sync_copy(dc_v, acc.at[ii_v], add=True)
                return 0

            lax.fori_loop(0, nsup, body, 0)
            return 0

        lax.fori_loop(0, ngrp, grp, 0)
        plsc.subcore_barrier()
        _acc_dump(acc, out_hbm, c, s, n, rpt, extra_off, extra)

    k = pl.kernel(
        body_fn,
        out_type=jax.ShapeDtypeStruct((2 * n, F), jnp.float32),
        mesh=_MESH,
        scratch_types=scratch,
        name=f"sc_pass_v{axis}" + ("" if has_v else "_nov"),
    )
    args = [nbr_i, nbr_j, phiA, phiC, DA, DCU]
    if has_v:
        args.append(va)
    args.append(zeros128)
    return k(*args)


def kernel(cg_xyz, CG_nbr_list, cg_s, params):
    n = cg_s.shape[0]
    nbr = jnp.concatenate([CG_nbr_list, CG_nbr_list[:, ::-1]], axis=0)
    nbr_i = nbr[:, 0] + 0
    nbr_j = nbr[:, 1] + 0
    n_orig = CG_nbr_list.shape[0]

    gt_f, lt_f = _directed_flags(CG_nbr_list)
    directed = jnp.logical_and(gt_f[0, 0] > 0, lt_f[0, 0] > 0)
    keep_second = jnp.where(directed, jnp.float32(0.0), jnp.float32(1.0))
    mask = jnp.concatenate([
        jnp.ones((n_orig,), jnp.float32),
        jnp.full((n_orig,), 1.0, jnp.float32) * keep_second,
    ])

    xyzpad = jnp.pad(cg_xyz, ((0, 0), (0, 125)))
    r16 = _xyz_gather(xyzpad, nbr_i, nbr_j)
    r = r16[:, :3]
    dist = jnp.sqrt((r * r + 1e-8).sum(-1))
    unit = r / dist[:, None]
    u16 = jnp.pad(unit, ((0, 0), (0, 13)))
    env = 0.5 * (jnp.cos(jnp.pi * dist / CUTOFF) + 1.0)
    env = jnp.where(dist <= CUTOFF, env, 0.0)
    envm = env * mask
    n_r = jnp.arange(1, NRBF + 1, dtype=jnp.float32)
    rbfenv = (jnp.sin(n_r * (jnp.pi / CUTOFF) * dist[:, None]) / dist[:, None]
              * envm[:, None])
    rbfenv = jnp.pad(rbfenv, ((0, 0), (0, NRBF_PAD - NRBF)))
    envm_col = envm[:, None]

    zeros128 = jnp.zeros((n, F), jnp.float32)

    d_panels = []
    for p in params:
        Wd_pad = jnp.pad(p["Wd"], ((0, NRBF_PAD - NRBF), (0, 0)))
        d_panels.append(_d_panels(rbfenv, envm_col, u16, Wd_pad, p["bd"][None, :]))

    s_cur = cg_s
    p0 = p1 = zeros128
    v_tabs = [None, None, None]
    for li, p in enumerate(params):
        s_cur, phiA, phiB, phiC = _phi_panels(
            s_cur, p0, p1, p["W1"], p["b1"], p["W2"], p["b2"])
        DA, DB, DCx, DCy, DCz = d_panels[li]
        DCU = (DCx, DCy, DCz)

        part_s = _sc_pass_s(nbr_i, nbr_j, phiB, DB, zeros128)
        p0, p1 = part_s[:n], part_s[n:]

        new_tabs = []
        for a in range(3):
            pv = _sc_pass_v(a, nbr_i, nbr_j, phiA, phiC, DA, DCU[a],
                            v_tabs[a], zeros128)
            old = zeros128 if v_tabs[a] is None else v_tabs[a]
            new_tabs.append(_merge3(old, pv[:n], pv[n:]))
        v_tabs = new_tabs

    s_out = _merge3(s_cur, p0, p1)
    v_out = jnp.stack(v_tabs, axis=-1)
    return (s_out, v_out)


# speedup vs baseline: 1.3909x; 1.3909x over previous
import functools

import jax
import jax.numpy as jnp
from jax import lax
from jax.experimental import pallas as pl
from jax.experimental.pallas import tpu as pltpu
from jax.experimental.pallas import tpu_sc as plsc

F = 128
NRBF = 20
NRBF_PAD = 24
CUTOFF = 5.0

NC = 2
NS = 16
NW = NC * NS
LANES = 16

_MESH = plsc.VectorSubcoreMesh(core_axis_name="c", subcore_axis_name="s")


def _flags_body(nbr_ref, gt_ref, lt_ref):
    blk = nbr_ref[...]
    gt = jnp.any(blk[:, 0] > blk[:, 1]).astype(jnp.float32)
    lt = jnp.any(blk[:, 1] > blk[:, 0]).astype(jnp.float32)

    @pl.when(pl.program_id(0) == 0)
    def _():
        gt_ref[...] = jnp.zeros_like(gt_ref)
        lt_ref[...] = jnp.zeros_like(lt_ref)

    gt_ref[...] = jnp.maximum(gt_ref[...], gt)
    lt_ref[...] = jnp.maximum(lt_ref[...], lt)


def _directed_flags(nbr_list):
    n = nbr_list.shape[0]
    rows = 8000
    grid = n // rows
    return pl.pallas_call(
        _flags_body,
        grid=(grid,),
        in_specs=[pl.BlockSpec((rows, 2), lambda i: (i, 0))],
        out_specs=[
            pl.BlockSpec((1, 128), lambda i: (0, 0)),
            pl.BlockSpec((1, 128), lambda i: (0, 0)),
        ],
        out_shape=[
            jax.ShapeDtypeStruct((1, 128), jnp.float32),
            jax.ShapeDtypeStruct((1, 128), jnp.float32),
        ],
        name="directed_flags",
    )(nbr_list)


def _phi_body(s_ref, p0_ref, p1_ref, w1_ref, b1_ref, w2_ref, b2_ref,
              snew_ref, a_ref, b_ref, c_ref):
    s = s_ref[...] + p0_ref[...] + p1_ref[...]
    snew_ref[...] = s
    h = jnp.dot(s, w1_ref[...], preferred_element_type=jnp.float32) + b1_ref[...]
    h = h * jax.nn.sigmoid(h)
    phi = jnp.dot(h, w2_ref[...], preferred_element_type=jnp.float32) + b2_ref[...]
    a_ref[...] = phi[:, 0:128]
    b_ref[...] = phi[:, 128:256]
    c_ref[...] = phi[:, 256:384]


def _phi_panels(s, p0, p1, W1, b1, W2, b2):
    n = s.shape[0]
    rows = 1000
    grid = n // rows
    return pl.pallas_call(
        _phi_body,
        grid=(grid,),
        in_specs=[
            pl.BlockSpec((rows, F), lambda i: (i, 0)),
            pl.BlockSpec((rows, F), lambda i: (i, 0)),
            pl.BlockSpec((rows, F), lambda i: (i, 0)),
            pl.BlockSpec((F, F), lambda i: (0, 0)),
            pl.BlockSpec((F,), lambda i: (0,)),
            pl.BlockSpec((F, 3 * F), lambda i: (0, 0)),
            pl.BlockSpec((3 * F,), lambda i: (0,)),
        ],
        out_specs=[
            pl.BlockSpec((rows, F), lambda i: (i, 0)),
            pl.BlockSpec((rows, F), lambda i: (i, 0)),
            pl.BlockSpec((rows, F), lambda i: (i, 0)),
            pl.BlockSpec((rows, F), lambda i: (i, 0)),
        ],
        out_shape=[
            jax.ShapeDtypeStruct((n, F), jnp.float32),
            jax.ShapeDtypeStruct((n, F), jnp.float32),
            jax.ShapeDtypeStruct((n, F), jnp.float32),
            jax.ShapeDtypeStruct((n, F), jnp.float32),
        ],
        name="phi_panels",
    )(s, p0, p1, W1, b1, W2, b2)


def _d_body(rbf_ref, envm_ref, u_ref, wd_ref, bd_ref,
            da_ref, db_ref, dcx_ref, dcy_ref, dcz_ref):
    d = jnp.dot(rbf_ref[...], wd_ref[...], preferred_element_type=jnp.float32)
    d = d + envm_ref[...] * bd_ref[...]
    da_ref[...] = d[:, 0:128]
    db_ref[...] = d[:, 128:256]
    dc = d[:, 256:384]
    dcx_ref[...] = dc * u_ref[:, 0:1]
    dcy_ref[...] = dc * u_ref[:, 1:2]
    dcz_ref[...] = dc * u_ref[:, 2:3]


def _d_panels(rbfenv, envm_col, u16, Wd_pad, bd):
    e = rbfenv.shape[0]
    rows = 512
    grid = e // rows
    return pl.pallas_call(
        _d_body,
        grid=(grid,),
        in_specs=[
            pl.BlockSpec((rows, NRBF_PAD), lambda i: (i, 0)),
            pl.BlockSpec((rows, 1), lambda i: (i, 0)),
            pl.BlockSpec((rows, 16), lambda i: (i, 0)),
            pl.BlockSpec((NRBF_PAD, 3 * F), lambda i: (0, 0)),
            pl.BlockSpec((1, 3 * F), lambda i: (0, 0)),
        ],
        out_specs=[pl.BlockSpec((rows, F), lambda i: (i, 0))] * 5,
        out_shape=[jax.ShapeDtypeStruct((e, F), jnp.float32)] * 5,
        name="d_panels",
    )(rbfenv, envm_col, u16, Wd_pad, bd)


def _merge3_body(a_ref, b_ref, c_ref, o_ref):
    o_ref[...] = a_ref[...] + b_ref[...] + c_ref[...]


def _merge3(a, b, c):
    n, k = a.shape
    rows = 1000
    grid = n // rows
    return pl.pallas_call(
        _merge3_body,
        grid=(grid,),
        in_specs=[pl.BlockSpec((rows, k), lambda i: (i, 0))] * 3,
        out_specs=pl.BlockSpec((rows, k), lambda i: (i, 0)),
        out_shape=jax.ShapeDtypeStruct((n, k), jnp.float32),
        name="merge3",
    )(a, b, c)


def _acc_zero(z_hbm, acc, s, rpt, extra_off, extra):
    r0 = s * rpt
    pltpu.sync_copy(z_hbm.at[pl.ds(r0, rpt), :], acc.at[pl.ds(r0, rpt), :])
    if extra:
        @pl.when(s == NS - 1)
        def _():
            pltpu.sync_copy(z_hbm.at[pl.ds(extra_off, extra), :],
                            acc.at[pl.ds(extra_off, extra), :])


def _acc_dump(acc, out_hbm, c, s, n, rpt, extra_off, extra):
    r0 = s * rpt
    pltpu.sync_copy(acc.at[pl.ds(r0, rpt), :],
                    out_hbm.at[pl.ds(c * n + r0, rpt), :])
    if extra:
        @pl.when(s == NS - 1)
        def _():
            pltpu.sync_copy(acc.at[pl.ds(extra_off, extra), :],
                            out_hbm.at[pl.ds(c * n + extra_off, extra), :])


def _row_split(n):
    rpt = (n // NS) // 8 * 8
    extra_off = rpt * NS
    extra = n - extra_off
    return rpt, extra_off, extra


def _xyz_gather(xyzpad, nbr_i, nbr_j):
    e2 = nbr_i.shape[0]
    epw = e2 // NW
    eb = 160
    nblk = epw // eb

    @functools.partial(
        pl.kernel,
        out_type=jax.ShapeDtypeStruct((e2, 16), jnp.float32),
        mesh=_MESH,
        scratch_types=[
            pltpu.VMEM((eb,), jnp.int32),
            pltpu.VMEM((eb,), jnp.int32),
            pltpu.VMEM((eb, 128), jnp.float32),
            pltpu.VMEM((eb, 128), jnp.float32),
            pltpu.VMEM((eb, 16), jnp.float32),
        ],
        name="sc_xyz_gather",
    )
    def k(xyz_hbm, ni_hbm, nj_hbm, r_hbm, ii_v, jj_v, gi_v, gj_v, o_v):
        c = lax.axis_index("c")
        s = lax.axis_index("s")
        wid = c * NS + s

        def body(b, _):
            off = wid * epw + b * eb
            pltpu.sync_copy(ni_hbm.at[pl.ds(off, eb)], ii_v)
            pltpu.sync_copy(nj_hbm.at[pl.ds(off, eb)], jj_v)
            pltpu.sync_copy(xyz_hbm.at[ii_v], gi_v)
            pltpu.sync_copy(xyz_hbm.at[jj_v], gj_v)

            def edge(ei, _):
                o_v[ei, :] = gj_v[ei, 0:16] - gi_v[ei, 0:16]
                return 0

            lax.fori_loop(0, eb, edge, 0)
            pltpu.sync_copy(o_v, r_hbm.at[pl.ds(off, eb), :])
            return 0

        lax.fori_loop(0, nblk, body, 0)

    return k(xyzpad, nbr_i, nbr_j)


def _sc_pass_s(nbr_i, nbr_j, phiB, DB, zeros128):
    n = phiB.shape[0]
    e2 = nbr_i.shape[0]
    epw = e2 // NW
    eb = 80
    nsup = 25
    nblk = epw // eb
    ngrp = nblk // nsup
    rpt, extra_off, extra = _row_split(n)

    @functools.partial(
        pl.kernel,
        out_type=jax.ShapeDtypeStruct((2 * n, F), jnp.float32),
        mesh=_MESH,
        scratch_types=[
            pltpu.VMEM_SHARED((n, F), jnp.float32),
            pltpu.VMEM((nsup * eb,), jnp.int32),
            pltpu.VMEM((eb,), jnp.int32),
            pltpu.VMEM((eb, F), jnp.float32),
            pltpu.VMEM((eb, F), jnp.float32),
            pltpu.SemaphoreType.DMA,
        ],
        name="sc_pass_s",
    )
    def k(ni_hbm, nj_hbm, phib_hbm, db_hbm, z_hbm, out_hbm,
          acc, jj_v, ii_v, gb_v, db_v, sem):
        c = lax.axis_index("c")
        s = lax.axis_index("s")
        wid = c * NS + s
        _acc_zero(z_hbm, acc, s, rpt, extra_off, extra)
        plsc.subcore_barrier()

        def grp(g, _):
            soff = wid * epw + g * nsup * eb
            pltpu.sync_copy(nj_hbm.at[pl.ds(soff, nsup * eb)], jj_v)

            def body(b, _):
                off = soff + b * eb
                ws = [
                    pltpu.async_copy(ni_hbm.at[pl.ds(off, eb)], ii_v, sem),
                    pltpu.async_copy(db_hbm.at[pl.ds(off, eb), :], db_v, sem),
                    pltpu.async_copy(
                        phib_hbm.at[jj_v.at[pl.ds(b * eb, eb)]], gb_v, sem),
                ]
                for w in ws:
                    w.wait()

                def edge(ei, _):
                    for kk in range(F // LANES):
                        sl = pl.ds(kk * LANES, LANES)
                        db_v[ei, sl] = db_v[ei, sl] * gb_v[ei, sl]
                    return 0

                lax.fori_loop(0, eb, edge, 0)
                pltpu.sync_copy(db_v, acc.at[ii_v], add=True)
                return 0

            lax.fori_loop(0, nsup, body, 0)
            return 0

        lax.fori_loop(0, ngrp, grp, 0)
        plsc.subcore_barrier()
        _acc_dump(acc, out_hbm, c, s, n, rpt, extra_off, extra)

    return k(nbr_i, nbr_j, phiB, DB, zeros128)


def _sc_pass_v(axis, nbr_i, nbr_j, phiA, phiC, DA, DCU, va, zeros128):
    n = phiA.shape[0]
    e2 = nbr_i.shape[0]
    epw = e2 // NW
    eb = 40
    nsup = 25
    nblk = epw // eb
    ngrp = nblk // nsup
    rpt, extra_off, extra = _row_split(n)
    has_v = va is not None

    scratch = [
        pltpu.VMEM_SHARED((n, F), jnp.float32),
        pltpu.VMEM((nsup * eb,), jnp.int32),
        pltpu.VMEM((eb,), jnp.int32),
        pltpu.VMEM((eb, F), jnp.float32),
        pltpu.VMEM((eb, F), jnp.float32),
        pltpu.SemaphoreType.DMA,
    ]
    if has_v:
        scratch += [
            pltpu.VMEM((eb, F), jnp.float32),
            pltpu.VMEM((eb, F), jnp.float32),
            pltpu.VMEM((eb, F), jnp.float32),
        ]

    def body_fn(*refs):
        if has_v:
            (ni_hbm, nj_hbm, pa_hbm, pc_hbm, da_hbm, dcu_hbm, va_hbm,
             z_hbm, out_hbm,
             acc, jj_v, ii_v, dc_v, gc_v, sem, da_v, ga_v, gv_v) = refs
        else:
            (ni_hbm, nj_hbm, pa_hbm, pc_hbm, da_hbm, dcu_hbm,
             z_hbm, out_hbm,
             acc, jj_v, ii_v, dc_v, gc_v, sem) = refs
            va_hbm = da_v = ga_v = gv_v = None
        c = lax.axis_index("c")
        s = lax.axis_index("s")
        wid = c * NS + s
        _acc_zero(z_hbm, acc, s, rpt, extra_off, extra)
        plsc.subcore_barrier()

        def grp(g, _):
            soff = wid * epw + g * nsup * eb
            pltpu.sync_copy(nj_hbm.at[pl.ds(soff, nsup * eb)], jj_v)

            def body(b, _):
                off = soff + b * eb
                jslice = jj_v.at[pl.ds(b * eb, eb)]
                ws = [
                    pltpu.async_copy(ni_hbm.at[pl.ds(off, eb)], ii_v, sem),
                    pltpu.async_copy(dcu_hbm.at[pl.ds(off, eb), :], dc_v, sem),
                    pltpu.async_copy(pc_hbm.at[jslice], gc_v, sem),
                ]
                if has_v:
                    ws += [
                        pltpu.async_copy(da_hbm.at[pl.ds(off, eb), :], da_v, sem),
                        pltpu.async_copy(pa_hbm.at[jslice], ga_v, sem),
                        pltpu.async_copy(va_hbm.at[jslice], gv_v, sem),
                    ]
                for w in ws:
                    w.wait()

                def edge(ei, _):
                    for kk in range(F // LANES):
                        sl = pl.ds(kk * LANES, LANES)
                        val = dc_v[ei, sl] * gc_v[ei, sl]
                        if has_v:
                            val = val + da_v[ei, sl] * ga_v[ei, sl] * gv_v[ei, sl]
                        dc_v[ei, sl] = val
                    return 0

                lax.fori_loop(0, eb, edge, 0)
                pltpu.sync_copy(dc_v, acc.at[ii_v], add=True)
                return 0

            lax.fori_loop(0, nsup, body, 0)
            return 0

        lax.fori_loop(0, ngrp, grp, 0)
        plsc.subcore_barrier()
        _acc_dump(acc, out_hbm, c, s, n, rpt, extra_off, extra)

    k = pl.kernel(
        body_fn,
        out_type=jax.ShapeDtypeStruct((2 * n, F), jnp.float32),
        mesh=_MESH,
        scratch_types=scratch,
        name=f"sc_pass_v{axis}" + ("" if has_v else "_nov"),
    )
    args = [nbr_i, nbr_j, phiA, phiC, DA, DCU]
    if has_v:
        args.append(va)
    args.append(zeros128)
    return k(*args)


def kernel(cg_xyz, CG_nbr_list, cg_s, params):
    n = cg_s.shape[0]
    nbr = jnp.concatenate([CG_nbr_list, CG_nbr_list[:, ::-1]], axis=0)
    nbr_i = nbr[:, 0] + 0
    nbr_j = nbr[:, 1] + 0
    n_orig = CG_nbr_list.shape[0]

    gt_f, lt_f = _directed_flags(CG_nbr_list)
    directed = jnp.logical_and(gt_f[0, 0] > 0, lt_f[0, 0] > 0)
    keep_second = jnp.where(directed, jnp.float32(0.0), jnp.float32(1.0))
    mask = jnp.concatenate([
        jnp.ones((n_orig,), jnp.float32),
        jnp.full((n_orig,), 1.0, jnp.float32) * keep_second,
    ])

    xyzpad = jnp.pad(cg_xyz, ((0, 0), (0, 125)))
    r16 = _xyz_gather(xyzpad, nbr_i, nbr_j)
    r = r16[:, :3]
    dist = jnp.sqrt((r * r + 1e-8).sum(-1))
    unit = r / dist[:, None]
    u16 = jnp.pad(unit, ((0, 0), (0, 13)))
    env = 0.5 * (jnp.cos(jnp.pi * dist / CUTOFF) + 1.0)
    env = jnp.where(dist <= CUTOFF, env, 0.0)
    envm = env * mask
    n_r = jnp.arange(1, NRBF + 1, dtype=jnp.float32)
    rbfenv = (jnp.sin(n_r * (jnp.pi / CUTOFF) * dist[:, None]) / dist[:, None]
              * envm[:, None])
    rbfenv = jnp.pad(rbfenv, ((0, 0), (0, NRBF_PAD - NRBF)))
    envm_col = envm[:, None]

    zeros128 = jnp.zeros((n, F), jnp.float32)

    d_panels = []
    for p in params:
        Wd_pad = jnp.pad(p["Wd"], ((0, NRBF_PAD - NRBF), (0, 0)))
        d_panels.append(_d_panels(rbfenv, envm_col, u16, Wd_pad, p["bd"][None, :]))

    s_cur = cg_s
    p0 = p1 = zeros128
    v_tabs = [None, None, None]
    for li, p in enumerate(params):
        s_cur, phiA, phiB, phiC = _phi_panels(
            s_cur, p0, p1, p["W1"], p["b1"], p["W2"], p["b2"])
        DA, DB, DCx, DCy, DCz = d_panels[li]
        DCU = (DCx, DCy, DCz)

        part_s = _sc_pass_s(nbr_i, nbr_j, phiB, DB, zeros128)
        p0, p1 = part_s[:n], part_s[n:]

        new_tabs = []
        for a in range(3):
            pv = _sc_pass_v(a, nbr_i, nbr_j, phiA, phiC, DA, DCU[a],
                            v_tabs[a], zeros128)
            old = zeros128 if v_tabs[a] is None else v_tabs[a]
            new_tabs.append(_merge3(old, pv[:n], pv[n:]))
        v_tabs = new_tabs

    s_out = _merge3(s_cur, p0, p1)
    v_out = jnp.stack(v_tabs, axis=-1)
    return (s_out, v_out)


# speedup vs baseline: 1.6478x; 1.1847x over previous
import functools

import jax
import jax.numpy as jnp
from jax import lax
from jax.experimental import pallas as pl
from jax.experimental.pallas import tpu as pltpu
from jax.experimental.pallas import tpu_sc as plsc

F = 128
NRBF = 20
NRBF_PAD = 24
CUTOFF = 5.0

NC = 2
NS = 16
NW = NC * NS
LANES = 16

_MESH = plsc.VectorSubcoreMesh(core_axis_name="c", subcore_axis_name="s")


def _flags_body(nbr_ref, gt_ref, lt_ref):
    blk = nbr_ref[...]
    gt = jnp.any(blk[:, 0] > blk[:, 1]).astype(jnp.float32)
    lt = jnp.any(blk[:, 1] > blk[:, 0]).astype(jnp.float32)

    @pl.when(pl.program_id(0) == 0)
    def _():
        gt_ref[...] = jnp.zeros_like(gt_ref)
        lt_ref[...] = jnp.zeros_like(lt_ref)

    gt_ref[...] = jnp.maximum(gt_ref[...], gt)
    lt_ref[...] = jnp.maximum(lt_ref[...], lt)


def _directed_flags(nbr_list):
    n = nbr_list.shape[0]
    rows = 8000
    grid = n // rows
    return pl.pallas_call(
        _flags_body,
        grid=(grid,),
        in_specs=[pl.BlockSpec((rows, 2), lambda i: (i, 0))],
        out_specs=[
            pl.BlockSpec((1, 128), lambda i: (0, 0)),
            pl.BlockSpec((1, 128), lambda i: (0, 0)),
        ],
        out_shape=[
            jax.ShapeDtypeStruct((1, 128), jnp.float32),
            jax.ShapeDtypeStruct((1, 128), jnp.float32),
        ],
        name="directed_flags",
    )(nbr_list)


def _phi_body(s_ref, p0_ref, p1_ref, w1_ref, b1_ref, w2_ref, b2_ref,
              snew_ref, a_ref, b_ref, c_ref):
    s = s_ref[...] + p0_ref[...] + p1_ref[...]
    snew_ref[...] = s
    h = jnp.dot(s, w1_ref[...], preferred_element_type=jnp.float32) + b1_ref[...]
    h = h * jax.nn.sigmoid(h)
    phi = jnp.dot(h, w2_ref[...], preferred_element_type=jnp.float32) + b2_ref[...]
    a_ref[...] = phi[:, 0:128]
    b_ref[...] = phi[:, 128:256]
    c_ref[...] = phi[:, 256:384]


def _phi_panels(s, p0, p1, W1, b1, W2, b2):
    n = s.shape[0]
    rows = 1000
    grid = n // rows
    return pl.pallas_call(
        _phi_body,
        grid=(grid,),
        in_specs=[
            pl.BlockSpec((rows, F), lambda i: (i, 0)),
            pl.BlockSpec((rows, F), lambda i: (i, 0)),
            pl.BlockSpec((rows, F), lambda i: (i, 0)),
            pl.BlockSpec((F, F), lambda i: (0, 0)),
            pl.BlockSpec((F,), lambda i: (0,)),
            pl.BlockSpec((F, 3 * F), lambda i: (0, 0)),
            pl.BlockSpec((3 * F,), lambda i: (0,)),
        ],
        out_specs=[
            pl.BlockSpec((rows, F), lambda i: (i, 0)),
            pl.BlockSpec((rows, F), lambda i: (i, 0)),
            pl.BlockSpec((rows, F), lambda i: (i, 0)),
            pl.BlockSpec((rows, F), lambda i: (i, 0)),
        ],
        out_shape=[
            jax.ShapeDtypeStruct((n, F), jnp.float32),
            jax.ShapeDtypeStruct((n, F), jnp.float32),
            jax.ShapeDtypeStruct((n, F), jnp.float32),
            jax.ShapeDtypeStruct((n, F), jnp.float32),
        ],
        name="phi_panels",
    )(s, p0, p1, W1, b1, W2, b2)


def _d_body(rbf_ref, envm_ref, u_ref, wd_ref, bd_ref,
            da_ref, db_ref, dcx_ref, dcy_ref, dcz_ref):
    d = jnp.dot(rbf_ref[...], wd_ref[...], preferred_element_type=jnp.float32)
    d = d + envm_ref[...] * bd_ref[...]
    da_ref[...] = d[:, 0:128]
    db_ref[...] = d[:, 128:256]
    dc = d[:, 256:384]
    dcx_ref[...] = dc * u_ref[:, 0:1]
    dcy_ref[...] = dc * u_ref[:, 1:2]
    dcz_ref[...] = dc * u_ref[:, 2:3]


def _d_panels(rbfenv, envm_col, u16, Wd_pad, bd):
    e = rbfenv.shape[0]
    rows = 512
    grid = e // rows
    return pl.pallas_call(
        _d_body,
        grid=(grid,),
        in_specs=[
            pl.BlockSpec((rows, NRBF_PAD), lambda i: (i, 0)),
            pl.BlockSpec((rows, 1), lambda i: (i, 0)),
            pl.BlockSpec((rows, 16), lambda i: (i, 0)),
            pl.BlockSpec((NRBF_PAD, 3 * F), lambda i: (0, 0)),
            pl.BlockSpec((1, 3 * F), lambda i: (0, 0)),
        ],
        out_specs=[pl.BlockSpec((rows, F), lambda i: (i, 0))] * 5,
        out_shape=[jax.ShapeDtypeStruct((e, F), jnp.float32)] * 5,
        name="d_panels",
    )(rbfenv, envm_col, u16, Wd_pad, bd)


def _mul2_body(a_ref, b_ref, o_ref):
    o_ref[...] = a_ref[...] * b_ref[...]


def _mul2(a, b):
    n, k = a.shape
    rows = 1000
    grid = n // rows
    return pl.pallas_call(
        _mul2_body,
        grid=(grid,),
        in_specs=[pl.BlockSpec((rows, k), lambda i: (i, 0))] * 2,
        out_specs=pl.BlockSpec((rows, k), lambda i: (i, 0)),
        out_shape=jax.ShapeDtypeStruct((n, k), jnp.float32),
        name="mul2",
    )(a, b)


def _merge3_body(a_ref, b_ref, c_ref, o_ref):
    o_ref[...] = a_ref[...] + b_ref[...] + c_ref[...]


def _merge3(a, b, c):
    n, k = a.shape
    rows = 1000
    grid = n // rows
    return pl.pallas_call(
        _merge3_body,
        grid=(grid,),
        in_specs=[pl.BlockSpec((rows, k), lambda i: (i, 0))] * 3,
        out_specs=pl.BlockSpec((rows, k), lambda i: (i, 0)),
        out_shape=jax.ShapeDtypeStruct((n, k), jnp.float32),
        name="merge3",
    )(a, b, c)


def _acc_zero(z_hbm, acc, s, rpt, extra_off, extra):
    r0 = s * rpt
    pltpu.sync_copy(z_hbm.at[pl.ds(r0, rpt), :], acc.at[pl.ds(r0, rpt), :])
    if extra:
        @pl.when(s == NS - 1)
        def _():
            pltpu.sync_copy(z_hbm.at[pl.ds(extra_off, extra), :],
                            acc.at[pl.ds(extra_off, extra), :])


def _acc_dump(acc, out_hbm, c, s, n, rpt, extra_off, extra):
    r0 = s * rpt
    pltpu.sync_copy(acc.at[pl.ds(r0, rpt), :],
                    out_hbm.at[pl.ds(c * n + r0, rpt), :])
    if extra:
        @pl.when(s == NS - 1)
        def _():
            pltpu.sync_copy(acc.at[pl.ds(extra_off, extra), :],
                            out_hbm.at[pl.ds(c * n + extra_off, extra), :])


def _row_split(n):
    rpt = (n // NS) // 8 * 8
    extra_off = rpt * NS
    extra = n - extra_off
    return rpt, extra_off, extra


def _xyz_gather(xyzpad, nbr_i, nbr_j):
    e2 = nbr_i.shape[0]
    epw = e2 // NW
    eb = 160
    nblk = epw // eb

    @functools.partial(
        pl.kernel,
        out_type=jax.ShapeDtypeStruct((e2, 16), jnp.float32),
        mesh=_MESH,
        scratch_types=[
            pltpu.VMEM((eb,), jnp.int32),
            pltpu.VMEM((eb,), jnp.int32),
            pltpu.VMEM((eb, 128), jnp.float32),
            pltpu.VMEM((eb, 128), jnp.float32),
            pltpu.VMEM((eb, 16), jnp.float32),
        ],
        name="sc_xyz_gather",
    )
    def k(xyz_hbm, ni_hbm, nj_hbm, r_hbm, ii_v, jj_v, gi_v, gj_v, o_v):
        c = lax.axis_index("c")
        s = lax.axis_index("s")
        wid = c * NS + s

        def body(b, _):
            off = wid * epw + b * eb
            pltpu.sync_copy(ni_hbm.at[pl.ds(off, eb)], ii_v)
            pltpu.sync_copy(nj_hbm.at[pl.ds(off, eb)], jj_v)
            pltpu.sync_copy(xyz_hbm.at[ii_v], gi_v)
            pltpu.sync_copy(xyz_hbm.at[jj_v], gj_v)

            def edge(ei, _):
                o_v[ei, :] = gj_v[ei, 0:16] - gi_v[ei, 0:16]
                return 0

            lax.fori_loop(0, eb, edge, 0)
            pltpu.sync_copy(o_v, r_hbm.at[pl.ds(off, eb), :])
            return 0

        lax.fori_loop(0, nblk, body, 0)

    return k(xyzpad, nbr_i, nbr_j)


def _sc_pass_s(nbr_i, nbr_j, phiB, DB, zeros128):
    n = phiB.shape[0]
    e2 = nbr_i.shape[0]
    epw = e2 // NW
    eb = 80
    nsup = 25
    nblk = epw // eb
    ngrp = nblk // nsup
    rpt, extra_off, extra = _row_split(n)

    @functools.partial(
        pl.kernel,
        out_type=jax.ShapeDtypeStruct((2 * n, F), jnp.float32),
        mesh=_MESH,
        scratch_types=[
            pltpu.VMEM_SHARED((n, F), jnp.float32),
            pltpu.VMEM((nsup * eb,), jnp.int32),
            pltpu.VMEM((eb,), jnp.int32),
            pltpu.VMEM((eb, F), jnp.float32),
            pltpu.VMEM((eb, F), jnp.float32),
            pltpu.SemaphoreType.DMA,
        ],
        name="sc_pass_s",
    )
    def k(ni_hbm, nj_hbm, phib_hbm, db_hbm, z_hbm, out_hbm,
          acc, jj_v, ii_v, gb_v, db_v, sem):
        c = lax.axis_index("c")
        s = lax.axis_index("s")
        wid = c * NS + s
        _acc_zero(z_hbm, acc, s, rpt, extra_off, extra)
        plsc.subcore_barrier()

        def grp(g, _):
            soff = wid * epw + g * nsup * eb
            pltpu.sync_copy(nj_hbm.at[pl.ds(soff, nsup * eb)], jj_v)

            def body(b, _):
                off = soff + b * eb
                ws = [
                    pltpu.async_copy(ni_hbm.at[pl.ds(off, eb)], ii_v, sem),
                    pltpu.async_copy(db_hbm.at[pl.ds(off, eb), :], db_v, sem),
                    pltpu.async_copy(
                        phib_hbm.at[jj_v.at[pl.ds(b * eb, eb)]], gb_v, sem),
                ]
                for w in ws:
                    w.wait()

                def edge(ei, _):
                    for kk in range(F // LANES):
                        sl = pl.ds(kk * LANES, LANES)
                        db_v[ei, sl] = db_v[ei, sl] * gb_v[ei, sl]
                    return 0

                lax.fori_loop(0, eb, edge, 0)
                pltpu.sync_copy(db_v, acc.at[ii_v], add=True)
                return 0

            lax.fori_loop(0, nsup, body, 0)
            return 0

        lax.fori_loop(0, ngrp, grp, 0)
        plsc.subcore_barrier()
        _acc_dump(acc, out_hbm, c, s, n, rpt, extra_off, extra)

    return k(nbr_i, nbr_j, phiB, DB, zeros128)


def _sc_pass_v(axis, nbr_i, nbr_j, phiC, DCU, DA, AV, zeros128):
    n = phiC.shape[0]
    e2 = nbr_i.shape[0]
    epw = e2 // NW
    eb = 80
    nsup = 25
    nblk = epw // eb
    ngrp = nblk // nsup
    rpt, extra_off, extra = _row_split(n)
    has_v = AV is not None

    scratch = [
        pltpu.VMEM_SHARED((n, F), jnp.float32),
        pltpu.VMEM((nsup * eb,), jnp.int32),
        pltpu.VMEM((eb,), jnp.int32),
        pltpu.VMEM((eb, F), jnp.float32),
        pltpu.VMEM((eb, F), jnp.float32),
        pltpu.SemaphoreType.DMA,
    ]
    if has_v:
        scratch += [
            pltpu.VMEM((eb, F), jnp.float32),
            pltpu.VMEM((eb, F), jnp.float32),
        ]

    def body_fn(*refs):
        if has_v:
            (ni_hbm, nj_hbm, pc_hbm, dcu_hbm, da_hbm, av_hbm,
             z_hbm, out_hbm,
             acc, jj_v, ii_v, dc_v, gc_v, sem, da_v, gav_v) = refs
        else:
            (ni_hbm, nj_hbm, pc_hbm, dcu_hbm,
             z_hbm, out_hbm,
             acc, jj_v, ii_v, dc_v, gc_v, sem) = refs
            da_hbm = av_hbm = da_v = gav_v = None
        c = lax.axis_index("c")
        s = lax.axis_index("s")
        wid = c * NS + s
        _acc_zero(z_hbm, acc, s, rpt, extra_off, extra)
        plsc.subcore_barrier()

        def grp(g, _):
            soff = wid * epw + g * nsup * eb
            pltpu.sync_copy(nj_hbm.at[pl.ds(soff, nsup * eb)], jj_v)

            def body(b, _):
                off = soff + b * eb
                jslice = jj_v.at[pl.ds(b * eb, eb)]
                ws = [
                    pltpu.async_copy(ni_hbm.at[pl.ds(off, eb)], ii_v, sem),
                    pltpu.async_copy(dcu_hbm.at[pl.ds(off, eb), :], dc_v, sem),
                    pltpu.async_copy(pc_hbm.at[jslice], gc_v, sem),
                ]
                if has_v:
                    ws += [
                        pltpu.async_copy(da_hbm.at[pl.ds(off, eb), :], da_v, sem),
                        pltpu.async_copy(av_hbm.at[jslice], gav_v, sem),
                    ]
                for w in ws:
                    w.wait()

                def edge(ei, _):
                    for kk in range(F // LANES):
                        sl = pl.ds(kk * LANES, LANES)
                        val = dc_v[ei, sl] * gc_v[ei, sl]
                        if has_v:
                            val = val + da_v[ei, sl] * gav_v[ei, sl]
                        dc_v[ei, sl] = val
                    return 0

                lax.fori_loop(0, eb, edge, 0)
                pltpu.sync_copy(dc_v, acc.at[ii_v], add=True)
                return 0

            lax.fori_loop(0, nsup, body, 0)
            return 0

        lax.fori_loop(0, ngrp, grp, 0)
        plsc.subcore_barrier()
        _acc_dump(acc, out_hbm, c, s, n, rpt, extra_off, extra)

    k = pl.kernel(
        body_fn,
        out_type=jax.ShapeDtypeStruct((2 * n, F), jnp.float32),
        mesh=_MESH,
        scratch_types=scratch,
        name=f"sc_pass_v{axis}" + ("" if has_v else "_nov"),
    )
    args = [nbr_i, nbr_j, phiC, DCU]
    if has_v:
        args += [DA, AV]
    args.append(zeros128)
    return k(*args)


def kernel(cg_xyz, CG_nbr_list, cg_s, params):
    n = cg_s.shape[0]
    nbr = jnp.concatenate([CG_nbr_list, CG_nbr_list[:, ::-1]], axis=0)
    nbr_i = nbr[:, 0] + 0
    nbr_j = nbr[:, 1] + 0
    n_orig = CG_nbr_list.shape[0]

    gt_f, lt_f = _directed_flags(CG_nbr_list)
    directed = jnp.logical_and(gt_f[0, 0] > 0, lt_f[0, 0] > 0)
    keep_second = jnp.where(directed, jnp.float32(0.0), jnp.float32(1.0))
    mask = jnp.concatenate([
        jnp.ones((n_orig,), jnp.float32),
        jnp.full((n_orig,), 1.0, jnp.float32) * keep_second,
    ])

    xyzpad = jnp.pad(cg_xyz, ((0, 0), (0, 125)))
    r16 = _xyz_gather(xyzpad, nbr_i, nbr_j)
    r = r16[:, :3]
    dist = jnp.sqrt((r * r + 1e-8).sum(-1))
    unit = r / dist[:, None]
    u16 = jnp.pad(unit, ((0, 0), (0, 13)))
    env = 0.5 * (jnp.cos(jnp.pi * dist / CUTOFF) + 1.0)
    env = jnp.where(dist <= CUTOFF, env, 0.0)
    envm = env * mask
    n_r = jnp.arange(1, NRBF + 1, dtype=jnp.float32)
    rbfenv = (jnp.sin(n_r * (jnp.pi / CUTOFF) * dist[:, None]) / dist[:, None]
              * envm[:, None])
    rbfenv = jnp.pad(rbfenv, ((0, 0), (0, NRBF_PAD - NRBF)))
    envm_col = envm[:, None]

    zeros128 = jnp.zeros((n, F), jnp.float32)

    d_panels = []
    for p in params:
        Wd_pad = jnp.pad(p["Wd"], ((0, NRBF_PAD - NRBF), (0, 0)))
        d_panels.append(_d_panels(rbfenv, envm_col, u16, Wd_pad, p["bd"][None, :]))

    s_cur = cg_s
    p0 = p1 = zeros128
    v_tabs = [None, None, None]
    for li, p in enumerate(params):
        s_cur, phiA, phiB, phiC = _phi_panels(
            s_cur, p0, p1, p["W1"], p["b1"], p["W2"], p["b2"])
        DA, DB, DCx, DCy, DCz = d_panels[li]
        DCU = (DCx, DCy, DCz)

        part_s = _sc_pass_s(nbr_i, nbr_j, phiB, DB, zeros128)
        p0, p1 = part_s[:n], part_s[n:]

        new_tabs = []
        for a in range(3):
            av = None if v_tabs[a] is None else _mul2(phiA, v_tabs[a])
            pv = _sc_pass_v(a, nbr_i, nbr_j, phiC, DCU[a], DA, av, zeros128)
            old = zeros128 if v_tabs[a] is None else v_tabs[a]
            new_tabs.append(_merge3(old, pv[:n], pv[n:]))
        v_tabs = new_tabs

    s_out = _merge3(s_cur, p0, p1)
    v_out = jnp.stack(v_tabs, axis=-1)
    return (s_out, v_out)


# speedup vs baseline: 1.7902x; 1.0864x over previous
import functools

import jax
import jax.numpy as jnp
from jax import lax
from jax.experimental import pallas as pl
from jax.experimental.pallas import tpu as pltpu
from jax.experimental.pallas import tpu_sc as plsc

F = 128
NRBF = 20
NRBF_PAD = 24
CUTOFF = 5.0

NC = 2
NS = 16
NW = NC * NS
LANES = 16

_MESH = plsc.VectorSubcoreMesh(core_axis_name="c", subcore_axis_name="s")


def _flags_body(nbr_ref, gt_ref, lt_ref):
    blk = nbr_ref[...]
    gt = jnp.any(blk[:, 0] > blk[:, 1]).astype(jnp.float32)
    lt = jnp.any(blk[:, 1] > blk[:, 0]).astype(jnp.float32)

    @pl.when(pl.program_id(0) == 0)
    def _():
        gt_ref[...] = jnp.zeros_like(gt_ref)
        lt_ref[...] = jnp.zeros_like(lt_ref)

    gt_ref[...] = jnp.maximum(gt_ref[...], gt)
    lt_ref[...] = jnp.maximum(lt_ref[...], lt)


def _directed_flags(nbr_list):
    n = nbr_list.shape[0]
    rows = 8000
    grid = n // rows
    return pl.pallas_call(
        _flags_body,
        grid=(grid,),
        in_specs=[pl.BlockSpec((rows, 2), lambda i: (i, 0))],
        out_specs=[
            pl.BlockSpec((1, 128), lambda i: (0, 0)),
            pl.BlockSpec((1, 128), lambda i: (0, 0)),
        ],
        out_shape=[
            jax.ShapeDtypeStruct((1, 128), jnp.float32),
            jax.ShapeDtypeStruct((1, 128), jnp.float32),
        ],
        name="directed_flags",
    )(nbr_list)


def _phi_body(s_ref, p0_ref, p1_ref, w1_ref, b1_ref, w2_ref, b2_ref,
              snew_ref, a_ref, b_ref, c_ref):
    s = s_ref[...] + p0_ref[...] + p1_ref[...]
    snew_ref[...] = s
    h = jnp.dot(s, w1_ref[...], preferred_element_type=jnp.float32) + b1_ref[...]
    h = h * jax.nn.sigmoid(h)
    phi = jnp.dot(h, w2_ref[...], preferred_element_type=jnp.float32) + b2_ref[...]
    a_ref[...] = phi[:, 0:128]
    b_ref[...] = phi[:, 128:256]
    c_ref[...] = phi[:, 256:384]


def _phi_panels(s, p0, p1, W1, b1, W2, b2):
    n = s.shape[0]
    rows = 1000
    grid = n // rows
    return pl.pallas_call(
        _phi_body,
        grid=(grid,),
        in_specs=[
            pl.BlockSpec((rows, F), lambda i: (i, 0)),
            pl.BlockSpec((rows, F), lambda i: (i, 0)),
            pl.BlockSpec((rows, F), lambda i: (i, 0)),
            pl.BlockSpec((F, F), lambda i: (0, 0)),
            pl.BlockSpec((F,), lambda i: (0,)),
            pl.BlockSpec((F, 3 * F), lambda i: (0, 0)),
            pl.BlockSpec((3 * F,), lambda i: (0,)),
        ],
        out_specs=[
            pl.BlockSpec((rows, F), lambda i: (i, 0)),
            pl.BlockSpec((rows, F), lambda i: (i, 0)),
            pl.BlockSpec((rows, F), lambda i: (i, 0)),
            pl.BlockSpec((rows, F), lambda i: (i, 0)),
        ],
        out_shape=[
            jax.ShapeDtypeStruct((n, F), jnp.float32),
            jax.ShapeDtypeStruct((n, F), jnp.float32),
            jax.ShapeDtypeStruct((n, F), jnp.float32),
            jax.ShapeDtypeStruct((n, F), jnp.float32),
        ],
        name="phi_panels",
    )(s, p0, p1, W1, b1, W2, b2)


def _d_body(rbf_ref, envm_ref, u_ref, wd_ref, bd_ref,
            da_ref, db_ref, dcx_ref, dcy_ref, dcz_ref):
    d = jnp.dot(rbf_ref[...], wd_ref[...], preferred_element_type=jnp.float32)
    d = d + envm_ref[...] * bd_ref[...]
    da_ref[...] = d[:, 0:128]
    db_ref[...] = d[:, 128:256]
    dc = d[:, 256:384]
    dcx_ref[...] = dc * u_ref[:, 0:1]
    dcy_ref[...] = dc * u_ref[:, 1:2]
    dcz_ref[...] = dc * u_ref[:, 2:3]


def _d_panels(rbfenv, envm_col, u16, Wd_pad, bd):
    e = rbfenv.shape[0]
    rows = 512
    grid = e // rows
    return pl.pallas_call(
        _d_body,
        grid=(grid,),
        in_specs=[
            pl.BlockSpec((rows, NRBF_PAD), lambda i: (i, 0)),
            pl.BlockSpec((rows, 1), lambda i: (i, 0)),
            pl.BlockSpec((rows, 16), lambda i: (i, 0)),
            pl.BlockSpec((NRBF_PAD, 3 * F), lambda i: (0, 0)),
            pl.BlockSpec((1, 3 * F), lambda i: (0, 0)),
        ],
        out_specs=[pl.BlockSpec((rows, F), lambda i: (i, 0))] * 5,
        out_shape=[jax.ShapeDtypeStruct((e, F), jnp.float32)] * 5,
        name="d_panels",
    )(rbfenv, envm_col, u16, Wd_pad, bd)


def _mul2_body(a_ref, b_ref, o_ref):
    o_ref[...] = a_ref[...] * b_ref[...]


def _mul2(a, b):
    n, k = a.shape
    rows = 1000
    grid = n // rows
    return pl.pallas_call(
        _mul2_body,
        grid=(grid,),
        in_specs=[pl.BlockSpec((rows, k), lambda i: (i, 0))] * 2,
        out_specs=pl.BlockSpec((rows, k), lambda i: (i, 0)),
        out_shape=jax.ShapeDtypeStruct((n, k), jnp.float32),
        name="mul2",
    )(a, b)


def _merge3_body(a_ref, b_ref, c_ref, o_ref):
    o_ref[...] = a_ref[...] + b_ref[...] + c_ref[...]


def _merge3(a, b, c):
    n, k = a.shape
    rows = 1000
    grid = n // rows
    return pl.pallas_call(
        _merge3_body,
        grid=(grid,),
        in_specs=[pl.BlockSpec((rows, k), lambda i: (i, 0))] * 3,
        out_specs=pl.BlockSpec((rows, k), lambda i: (i, 0)),
        out_shape=jax.ShapeDtypeStruct((n, k), jnp.float32),
        name="merge3",
    )(a, b, c)


def _acc_zero(z_hbm, acc, s, rpt, extra_off, extra):
    r0 = s * rpt
    pltpu.sync_copy(z_hbm.at[pl.ds(r0, rpt), :], acc.at[pl.ds(r0, rpt), :])
    if extra:
        @pl.when(s == NS - 1)
        def _():
            pltpu.sync_copy(z_hbm.at[pl.ds(extra_off, extra), :],
                            acc.at[pl.ds(extra_off, extra), :])


def _acc_dump(acc, out_hbm, c, s, n, rpt, extra_off, extra):
    r0 = s * rpt
    pltpu.sync_copy(acc.at[pl.ds(r0, rpt), :],
                    out_hbm.at[pl.ds(c * n + r0, rpt), :])
    if extra:
        @pl.when(s == NS - 1)
        def _():
            pltpu.sync_copy(acc.at[pl.ds(extra_off, extra), :],
                            out_hbm.at[pl.ds(c * n + extra_off, extra), :])


def _row_split(n):
    rpt = (n // NS) // 8 * 8
    extra_off = rpt * NS
    extra = n - extra_off
    return rpt, extra_off, extra


def _xyz_gather(xyzpad, nbr_i, nbr_j):
    e2 = nbr_i.shape[0]
    epw = e2 // NW
    eb = 160
    nblk = epw // eb

    @functools.partial(
        pl.kernel,
        out_type=jax.ShapeDtypeStruct((e2, 16), jnp.float32),
        mesh=_MESH,
        scratch_types=[
            pltpu.VMEM((eb,), jnp.int32),
            pltpu.VMEM((eb,), jnp.int32),
            pltpu.VMEM((eb, 128), jnp.float32),
            pltpu.VMEM((eb, 128), jnp.float32),
            pltpu.VMEM((eb, 16), jnp.float32),
        ],
        name="sc_xyz_gather",
    )
    def k(xyz_hbm, ni_hbm, nj_hbm, r_hbm, ii_v, jj_v, gi_v, gj_v, o_v):
        c = lax.axis_index("c")
        s = lax.axis_index("s")
        wid = c * NS + s

        def body(b, _):
            off = wid * epw + b * eb
            pltpu.sync_copy(ni_hbm.at[pl.ds(off, eb)], ii_v)
            pltpu.sync_copy(nj_hbm.at[pl.ds(off, eb)], jj_v)
            pltpu.sync_copy(xyz_hbm.at[ii_v], gi_v)
            pltpu.sync_copy(xyz_hbm.at[jj_v], gj_v)

            def edge(ei, _):
                o_v[ei, :] = gj_v[ei, 0:16] - gi_v[ei, 0:16]
                return 0

            lax.fori_loop(0, eb, edge, 0)
            pltpu.sync_copy(o_v, r_hbm.at[pl.ds(off, eb), :])
            return 0

        lax.fori_loop(0, nblk, body, 0)

    return k(xyzpad, nbr_i, nbr_j)


def _sc_pass_s(nbr_i, nbr_j, phiB, DB, zeros128):
    n = phiB.shape[0]
    e2 = nbr_i.shape[0]
    epw = e2 // NW
    eb = 80
    nsup = 25
    nblk = epw // eb
    ngrp = nblk // nsup
    rpt, extra_off, extra = _row_split(n)

    @functools.partial(
        pl.kernel,
        out_type=jax.ShapeDtypeStruct((2 * n, F), jnp.float32),
        mesh=_MESH,
        scratch_types=[
            pltpu.VMEM_SHARED((n, F), jnp.float32),
            pltpu.VMEM((nsup * eb,), jnp.int32),
            pltpu.VMEM((eb,), jnp.int32),
            pltpu.VMEM((eb, F), jnp.float32),
            pltpu.VMEM((eb, F), jnp.float32),
            pltpu.SemaphoreType.DMA,
        ],
        name="sc_pass_s",
    )
    def k(ni_hbm, nj_hbm, phib_hbm, db_hbm, z_hbm, out_hbm,
          acc, jj_v, ii_v, gb_v, db_v, sem):
        c = lax.axis_index("c")
        s = lax.axis_index("s")
        wid = c * NS + s
        _acc_zero(z_hbm, acc, s, rpt, extra_off, extra)
        plsc.subcore_barrier()

        def grp(g, _):
            soff = wid * epw + g * nsup * eb
            pltpu.sync_copy(nj_hbm.at[pl.ds(soff, nsup * eb)], jj_v)

            def body(b, _):
                off = soff + b * eb
                ws = [
                    pltpu.async_copy(ni_hbm.at[pl.ds(off, eb)], ii_v, sem),
                    pltpu.async_copy(db_hbm.at[pl.ds(off, eb), :], db_v, sem),
                    pltpu.async_copy(
                        phib_hbm.at[jj_v.at[pl.ds(b * eb, eb)]], gb_v, sem),
                ]
                for w in ws:
                    w.wait()

                def edge(ei, _):
                    for kk in range(F // LANES):
                        sl = pl.ds(kk * LANES, LANES)
                        db_v[ei, sl] = db_v[ei, sl] * gb_v[ei, sl]
                    return 0

                lax.fori_loop(0, eb, edge, 0)
                pltpu.sync_copy(db_v, acc.at[ii_v], add=True)
                return 0

            lax.fori_loop(0, nsup, body, 0)
            return 0

        lax.fori_loop(0, ngrp, grp, 0)
        plsc.subcore_barrier()
        _acc_dump(acc, out_hbm, c, s, n, rpt, extra_off, extra)

    return k(nbr_i, nbr_j, phiB, DB, zeros128)


def _sc_pass_v(axis, nbr_i, nbr_j, phiC, DCU, DA, AV, zeros128):
    n = phiC.shape[0]
    e2 = nbr_i.shape[0]
    epw = e2 // NW
    eb = 40
    nsup = 10
    nblk = epw // eb
    ngrp = nblk // nsup
    npair = nsup // 2
    rpt, extra_off, extra = _row_split(n)
    has_v = AV is not None

    scratch = [
        pltpu.VMEM_SHARED((n, F), jnp.float32),
        pltpu.VMEM((nsup * eb,), jnp.int32),
        pltpu.VMEM((2, eb), jnp.int32),
        pltpu.VMEM((2 * eb, F), jnp.float32),
        pltpu.VMEM((2 * eb, F), jnp.float32),
        pltpu.SemaphoreType.DMA,
        pltpu.SemaphoreType.DMA,
        pltpu.SemaphoreType.DMA,
    ]
    if has_v:
        scratch += [
            pltpu.VMEM((2 * eb, F), jnp.float32),
            pltpu.VMEM((2 * eb, F), jnp.float32),
        ]

    def body_fn(*refs):
        if has_v:
            (ni_hbm, nj_hbm, pc_hbm, dcu_hbm, da_hbm, av_hbm,
             z_hbm, out_hbm,
             acc, jj_v, ii_v, dc_v, gc_v, semw, sems0, sems1,
             da_v, gav_v) = refs
        else:
            (ni_hbm, nj_hbm, pc_hbm, dcu_hbm,
             z_hbm, out_hbm,
             acc, jj_v, ii_v, dc_v, gc_v, semw, sems0, sems1) = refs
            da_hbm = av_hbm = da_v = gav_v = None
        sems = (sems0, sems1)
        c = lax.axis_index("c")
        s = lax.axis_index("s")
        wid = c * NS + s
        _acc_zero(z_hbm, acc, s, rpt, extra_off, extra)
        plsc.subcore_barrier()

        def fire_wave(off, b, par):
            row = pl.ds(par * eb, eb)
            jslice = jj_v.at[pl.ds(b * eb, eb)]
            pltpu.async_copy(ni_hbm.at[pl.ds(off, eb)], ii_v.at[par], semw)
            pltpu.async_copy(dcu_hbm.at[pl.ds(off, eb), :],
                             dc_v.at[row, :], semw)
            pltpu.async_copy(pc_hbm.at[jslice], gc_v.at[row, :], semw)
            if has_v:
                pltpu.async_copy(da_hbm.at[pl.ds(off, eb), :],
                                 da_v.at[row, :], semw)
                pltpu.async_copy(av_hbm.at[jslice], gav_v.at[row, :], semw)

        def wait_wave(par):
            row = pl.ds(par * eb, eb)
            pltpu.make_async_copy(ni_hbm.at[pl.ds(0, eb)], ii_v.at[par],
                                  semw).wait()
            pltpu.make_async_copy(dcu_hbm.at[pl.ds(0, eb), :],
                                  dc_v.at[row, :], semw).wait()
            pltpu.make_async_copy(pc_hbm.at[pl.ds(0, eb), :],
                                  gc_v.at[row, :], semw).wait()
            if has_v:
                pltpu.make_async_copy(da_hbm.at[pl.ds(0, eb), :],
                                      da_v.at[row, :], semw).wait()
                pltpu.make_async_copy(av_hbm.at[pl.ds(0, eb), :],
                                      gav_v.at[row, :], semw).wait()

        def compute(par):
            base = par * eb

            def edge(ei, _):
                for kk in range(F // LANES):
                    sl = pl.ds(kk * LANES, LANES)
                    val = dc_v[base + ei, sl] * gc_v[base + ei, sl]
                    if has_v:
                        val = val + da_v[base + ei, sl] * gav_v[base + ei, sl]
                    dc_v[base + ei, sl] = val
                return 0

            lax.fori_loop(0, eb, edge, 0)

        def fire_scatter(par):
            row = pl.ds(par * eb, eb)
            pltpu.async_copy(dc_v.at[row, :], acc.at[ii_v.at[par]],
                             sems[par], add=True)

        def wait_scatter(par):
            row = pl.ds(par * eb, eb)
            pltpu.make_async_copy(dc_v.at[row, :], acc.at[ii_v.at[par]],
                                  sems[par]).wait()

        def grp(g, _):
            soff = wid * epw + g * nsup * eb
            pltpu.sync_copy(nj_hbm.at[pl.ds(soff, nsup * eb)], jj_v)
            fire_wave(soff, 0, 0)

            def pair(p, _):
                for par in range(2):
                    b = 2 * p + par
                    wait_wave(par)
                    nxt = b + 1

                    @pl.when(nxt < nsup)
                    def _():
                        fire_wave(soff + nxt * eb, nxt, 1 - par)

                    @pl.when(p > 0)
                    def _():
                        wait_scatter(par)

                    compute(par)
                    fire_scatter(par)
                return 0

            lax.fori_loop(0, npair, pair, 0)
            wait_scatter(0)
            wait_scatter(1)
            return 0

        lax.fori_loop(0, ngrp, grp, 0)
        plsc.subcore_barrier()
        _acc_dump(acc, out_hbm, c, s, n, rpt, extra_off, extra)

    k = pl.kernel(
        body_fn,
        out_type=jax.ShapeDtypeStruct((2 * n, F), jnp.float32),
        mesh=_MESH,
        scratch_types=scratch,
        name=f"sc_pass_v{axis}" + ("" if has_v else "_nov"),
    )
    args = [nbr_i, nbr_j, phiC, DCU]
    if has_v:
        args += [DA, AV]
    args.append(zeros128)
    return k(*args)


def kernel(cg_xyz, CG_nbr_list, cg_s, params):
    n = cg_s.shape[0]
    nbr = jnp.concatenate([CG_nbr_list, CG_nbr_list[:, ::-1]], axis=0)
    nbr_i = nbr[:, 0] + 0
    nbr_j = nbr[:, 1] + 0
    n_orig = CG_nbr_list.shape[0]

    gt_f, lt_f = _directed_flags(CG_nbr_list)
    directed = jnp.logical_and(gt_f[0, 0] > 0, lt_f[0, 0] > 0)
    keep_second = jnp.where(directed, jnp.float32(0.0), jnp.float32(1.0))
    mask = jnp.concatenate([
        jnp.ones((n_orig,), jnp.float32),
        jnp.full((n_orig,), 1.0, jnp.float32) * keep_second,
    ])

    xyzpad = jnp.pad(cg_xyz, ((0, 0), (0, 125)))
    r16 = _xyz_gather(xyzpad, nbr_i, nbr_j)
    r = r16[:, :3]
    dist = jnp.sqrt((r * r + 1e-8).sum(-1))
    unit = r / dist[:, None]
    u16 = jnp.pad(unit, ((0, 0), (0, 13)))
    env = 0.5 * (jnp.cos(jnp.pi * dist / CUTOFF) + 1.0)
    env = jnp.where(dist <= CUTOFF, env, 0.0)
    envm = env * mask
    n_r = jnp.arange(1, NRBF + 1, dtype=jnp.float32)
    rbfenv = (jnp.sin(n_r * (jnp.pi / CUTOFF) * dist[:, None]) / dist[:, None]
              * envm[:, None])
    rbfenv = jnp.pad(rbfenv, ((0, 0), (0, NRBF_PAD - NRBF)))
    envm_col = envm[:, None]

    zeros128 = jnp.zeros((n, F), jnp.float32)

    d_panels = []
    for p in params:
        Wd_pad = jnp.pad(p["Wd"], ((0, NRBF_PAD - NRBF), (0, 0)))
        d_panels.append(_d_panels(rbfenv, envm_col, u16, Wd_pad, p["bd"][None, :]))

    s_cur = cg_s
    p0 = p1 = zeros128
    v_tabs = [None, None, None]
    for li, p in enumerate(params):
        s_cur, phiA, phiB, phiC = _phi_panels(
            s_cur, p0, p1, p["W1"], p["b1"], p["W2"], p["b2"])
        DA, DB, DCx, DCy, DCz = d_panels[li]
        DCU = (DCx, DCy, DCz)

        part_s = _sc_pass_s(nbr_i, nbr_j, phiB, DB, zeros128)
        p0, p1 = part_s[:n], part_s[n:]

        new_tabs = []
        for a in range(3):
            av = None if v_tabs[a] is None else _mul2(phiA, v_tabs[a])
            pv = _sc_pass_v(a, nbr_i, nbr_j, phiC, DCU[a], DA, av, zeros128)
            old = zeros128 if v_tabs[a] is None else v_tabs[a]
            new_tabs.append(_merge3(old, pv[:n], pv[n:]))
        v_tabs = new_tabs

    s_out = _merge3(s_cur, p0, p1)
    v_out = jnp.stack(v_tabs, axis=-1)
    return (s_out, v_out)


# speedup vs baseline: 1.8036x; 1.0075x over previous
import functools

import jax
import jax.numpy as jnp
from jax import lax
from jax.experimental import pallas as pl
from jax.experimental.pallas import tpu as pltpu
from jax.experimental.pallas import tpu_sc as plsc

F = 128
NRBF = 20
NRBF_PAD = 24
CUTOFF = 5.0

NC = 2
NS = 16
NW = NC * NS
LANES = 16

_MESH = plsc.VectorSubcoreMesh(core_axis_name="c", subcore_axis_name="s")


def _flags_body(nbr_ref, gt_ref, lt_ref):
    blk = nbr_ref[...]
    gt = jnp.any(blk[:, 0] > blk[:, 1]).astype(jnp.float32)
    lt = jnp.any(blk[:, 1] > blk[:, 0]).astype(jnp.float32)

    @pl.when(pl.program_id(0) == 0)
    def _():
        gt_ref[...] = jnp.zeros_like(gt_ref)
        lt_ref[...] = jnp.zeros_like(lt_ref)

    gt_ref[...] = jnp.maximum(gt_ref[...], gt)
    lt_ref[...] = jnp.maximum(lt_ref[...], lt)


def _directed_flags(nbr_list):
    n = nbr_list.shape[0]
    rows = 8000
    grid = n // rows
    return pl.pallas_call(
        _flags_body,
        grid=(grid,),
        in_specs=[pl.BlockSpec((rows, 2), lambda i: (i, 0))],
        out_specs=[
            pl.BlockSpec((1, 128), lambda i: (0, 0)),
            pl.BlockSpec((1, 128), lambda i: (0, 0)),
        ],
        out_shape=[
            jax.ShapeDtypeStruct((1, 128), jnp.float32),
            jax.ShapeDtypeStruct((1, 128), jnp.float32),
        ],
        name="directed_flags",
    )(nbr_list)


def _phi_body(s_ref, p0_ref, p1_ref, w1_ref, b1_ref, w2_ref, b2_ref,
              snew_ref, a_ref, b_ref, c_ref):
    s = s_ref[...] + p0_ref[...] + p1_ref[...]
    snew_ref[...] = s
    h = jnp.dot(s, w1_ref[...], preferred_element_type=jnp.float32) + b1_ref[...]
    h = h * jax.nn.sigmoid(h)
    phi = jnp.dot(h, w2_ref[...], preferred_element_type=jnp.float32) + b2_ref[...]
    a_ref[...] = phi[:, 0:128]
    b_ref[...] = phi[:, 128:256]
    c_ref[...] = phi[:, 256:384]


def _phi_panels(s, p0, p1, W1, b1, W2, b2):
    n = s.shape[0]
    rows = 1000
    grid = n // rows
    return pl.pallas_call(
        _phi_body,
        grid=(grid,),
        in_specs=[
            pl.BlockSpec((rows, F), lambda i: (i, 0)),
            pl.BlockSpec((rows, F), lambda i: (i, 0)),
            pl.BlockSpec((rows, F), lambda i: (i, 0)),
            pl.BlockSpec((F, F), lambda i: (0, 0)),
            pl.BlockSpec((F,), lambda i: (0,)),
            pl.BlockSpec((F, 3 * F), lambda i: (0, 0)),
            pl.BlockSpec((3 * F,), lambda i: (0,)),
        ],
        out_specs=[
            pl.BlockSpec((rows, F), lambda i: (i, 0)),
            pl.BlockSpec((rows, F), lambda i: (i, 0)),
            pl.BlockSpec((rows, F), lambda i: (i, 0)),
            pl.BlockSpec((rows, F), lambda i: (i, 0)),
        ],
        out_shape=[
            jax.ShapeDtypeStruct((n, F), jnp.float32),
            jax.ShapeDtypeStruct((n, F), jnp.float32),
            jax.ShapeDtypeStruct((n, F), jnp.float32),
            jax.ShapeDtypeStruct((n, F), jnp.float32),
        ],
        name="phi_panels",
    )(s, p0, p1, W1, b1, W2, b2)


def _d_body(rbf_ref, envm_ref, u_ref, wd_ref, bd_ref,
            da_ref, db_ref, dcx_ref, dcy_ref, dcz_ref):
    d = jnp.dot(rbf_ref[...], wd_ref[...], preferred_element_type=jnp.float32)
    d = d + envm_ref[...] * bd_ref[...]
    da_ref[...] = d[:, 0:128]
    db_ref[...] = d[:, 128:256]
    dc = d[:, 256:384]
    dcx_ref[...] = dc * u_ref[:, 0:1]
    dcy_ref[...] = dc * u_ref[:, 1:2]
    dcz_ref[...] = dc * u_ref[:, 2:3]


def _d_panels(rbfenv, envm_col, u16, Wd_pad, bd):
    e = rbfenv.shape[0]
    rows = 512
    grid = e // rows
    return pl.pallas_call(
        _d_body,
        grid=(grid,),
        in_specs=[
            pl.BlockSpec((rows, NRBF_PAD), lambda i: (i, 0)),
            pl.BlockSpec((rows, 1), lambda i: (i, 0)),
            pl.BlockSpec((rows, 16), lambda i: (i, 0)),
            pl.BlockSpec((NRBF_PAD, 3 * F), lambda i: (0, 0)),
            pl.BlockSpec((1, 3 * F), lambda i: (0, 0)),
        ],
        out_specs=[pl.BlockSpec((rows, F), lambda i: (i, 0))] * 5,
        out_shape=[jax.ShapeDtypeStruct((e, F), jnp.float32)] * 5,
        name="d_panels",
    )(rbfenv, envm_col, u16, Wd_pad, bd)


def _mul2_body(a_ref, b_ref, o_ref):
    o_ref[...] = a_ref[...] * b_ref[...]


def _mul2(a, b):
    n, k = a.shape
    rows = 1000
    grid = n // rows
    return pl.pallas_call(
        _mul2_body,
        grid=(grid,),
        in_specs=[pl.BlockSpec((rows, k), lambda i: (i, 0))] * 2,
        out_specs=pl.BlockSpec((rows, k), lambda i: (i, 0)),
        out_shape=jax.ShapeDtypeStruct((n, k), jnp.float32),
        name="mul2",
    )(a, b)


def _merge3_body(a_ref, b_ref, c_ref, o_ref):
    o_ref[...] = a_ref[...] + b_ref[...] + c_ref[...]


def _merge3(a, b, c):
    n, k = a.shape
    rows = 1000
    grid = n // rows
    return pl.pallas_call(
        _merge3_body,
        grid=(grid,),
        in_specs=[pl.BlockSpec((rows, k), lambda i: (i, 0))] * 3,
        out_specs=pl.BlockSpec((rows, k), lambda i: (i, 0)),
        out_shape=jax.ShapeDtypeStruct((n, k), jnp.float32),
        name="merge3",
    )(a, b, c)


def _acc_zero(z_hbm, acc, s, rpt, extra_off, extra):
    r0 = s * rpt
    pltpu.sync_copy(z_hbm.at[pl.ds(r0, rpt), :], acc.at[pl.ds(r0, rpt), :])
    if extra:
        @pl.when(s == NS - 1)
        def _():
            pltpu.sync_copy(z_hbm.at[pl.ds(extra_off, extra), :],
                            acc.at[pl.ds(extra_off, extra), :])


def _acc_dump(acc, out_hbm, c, s, n, rpt, extra_off, extra):
    r0 = s * rpt
    pltpu.sync_copy(acc.at[pl.ds(r0, rpt), :],
                    out_hbm.at[pl.ds(c * n + r0, rpt), :])
    if extra:
        @pl.when(s == NS - 1)
        def _():
            pltpu.sync_copy(acc.at[pl.ds(extra_off, extra), :],
                            out_hbm.at[pl.ds(c * n + extra_off, extra), :])


def _row_split(n):
    rpt = (n // NS) // 8 * 8
    extra_off = rpt * NS
    extra = n - extra_off
    return rpt, extra_off, extra


def _xyz_gather(xyzpad, nbr_i, nbr_j):
    e2 = nbr_i.shape[0]
    epw = e2 // NW
    eb = 160
    nblk = epw // eb

    @functools.partial(
        pl.kernel,
        out_type=jax.ShapeDtypeStruct((e2, 16), jnp.float32),
        mesh=_MESH,
        scratch_types=[
            pltpu.VMEM((eb,), jnp.int32),
            pltpu.VMEM((eb,), jnp.int32),
            pltpu.VMEM((eb, 128), jnp.float32),
            pltpu.VMEM((eb, 128), jnp.float32),
            pltpu.VMEM((eb, 16), jnp.float32),
        ],
        name="sc_xyz_gather",
    )
    def k(xyz_hbm, ni_hbm, nj_hbm, r_hbm, ii_v, jj_v, gi_v, gj_v, o_v):
        c = lax.axis_index("c")
        s = lax.axis_index("s")
        wid = c * NS + s

        def body(b, _):
            off = wid * epw + b * eb
            pltpu.sync_copy(ni_hbm.at[pl.ds(off, eb)], ii_v)
            pltpu.sync_copy(nj_hbm.at[pl.ds(off, eb)], jj_v)
            pltpu.sync_copy(xyz_hbm.at[ii_v], gi_v)
            pltpu.sync_copy(xyz_hbm.at[jj_v], gj_v)

            def edge(ei, _):
                o_v[ei, :] = gj_v[ei, 0:16] - gi_v[ei, 0:16]
                return 0

            lax.fori_loop(0, eb, edge, 0)
            pltpu.sync_copy(o_v, r_hbm.at[pl.ds(off, eb), :])
            return 0

        lax.fori_loop(0, nblk, body, 0)

    return k(xyzpad, nbr_i, nbr_j)


def _sc_pass_s(nbr_i, nbr_j, phiB, DB, zeros128):
    n = phiB.shape[0]
    e2 = nbr_i.shape[0]
    epw = e2 // NW
    eb = 40
    nsup = 10
    nblk = epw // eb
    ngrp = nblk // nsup
    npair = nsup // 2
    rpt, extra_off, extra = _row_split(n)

    @functools.partial(
        pl.kernel,
        out_type=jax.ShapeDtypeStruct((2 * n, F), jnp.float32),
        mesh=_MESH,
        scratch_types=[
            pltpu.VMEM_SHARED((n, F), jnp.float32),
            pltpu.VMEM((nsup * eb,), jnp.int32),
            pltpu.VMEM((2, eb), jnp.int32),
            pltpu.VMEM((2 * eb, F), jnp.float32),
            pltpu.VMEM((2 * eb, F), jnp.float32),
            pltpu.SemaphoreType.DMA,
            pltpu.SemaphoreType.DMA,
            pltpu.SemaphoreType.DMA,
        ],
        name="sc_pass_s",
    )
    def k(ni_hbm, nj_hbm, phib_hbm, db_hbm, z_hbm, out_hbm,
          acc, jj_v, ii_v, db_v, gb_v, semw, sems0, sems1):
        sems = (sems0, sems1)
        c = lax.axis_index("c")
        s = lax.axis_index("s")
        wid = c * NS + s
        _acc_zero(z_hbm, acc, s, rpt, extra_off, extra)
        plsc.subcore_barrier()

        def fire_wave(off, b, par):
            row = pl.ds(par * eb, eb)
            jslice = jj_v.at[pl.ds(b * eb, eb)]
            pltpu.async_copy(ni_hbm.at[pl.ds(off, eb)], ii_v.at[par], semw)
            pltpu.async_copy(db_hbm.at[pl.ds(off, eb), :], db_v.at[row, :], semw)
            pltpu.async_copy(phib_hbm.at[jslice], gb_v.at[row, :], semw)

        def wait_wave(par):
            row = pl.ds(par * eb, eb)
            pltpu.make_async_copy(ni_hbm.at[pl.ds(0, eb)], ii_v.at[par],
                                  semw).wait()
            pltpu.make_async_copy(db_hbm.at[pl.ds(0, eb), :],
                                  db_v.at[row, :], semw).wait()
            pltpu.make_async_copy(phib_hbm.at[pl.ds(0, eb), :],
                                  gb_v.at[row, :], semw).wait()

        def compute(par):
            base = par * eb

            def edge(ei, _):
                for kk in range(F // LANES):
                    sl = pl.ds(kk * LANES, LANES)
                    db_v[base + ei, sl] = (db_v[base + ei, sl]
                                           * gb_v[base + ei, sl])
                return 0

            lax.fori_loop(0, eb, edge, 0)

        def fire_scatter(par):
            row = pl.ds(par * eb, eb)
            pltpu.async_copy(db_v.at[row, :], acc.at[ii_v.at[par]],
                             sems[par], add=True)

        def wait_scatter(par):
            row = pl.ds(par * eb, eb)
            pltpu.make_async_copy(db_v.at[row, :], acc.at[ii_v.at[par]],
                                  sems[par]).wait()

        def grp(g, _):
            soff = wid * epw + g * nsup * eb
            pltpu.sync_copy(nj_hbm.at[pl.ds(soff, nsup * eb)], jj_v)
            fire_wave(soff, 0, 0)

            def pair(p, _):
                for par in range(2):
                    b = 2 * p + par
                    wait_wave(par)
                    nxt = b + 1

                    @pl.when(nxt < nsup)
                    def _():
                        fire_wave(soff + nxt * eb, nxt, 1 - par)

                    @pl.when(p > 0)
                    def _():
                        wait_scatter(par)

                    compute(par)
                    fire_scatter(par)
                return 0

            lax.fori_loop(0, npair, pair, 0)
            wait_scatter(0)
            wait_scatter(1)
            return 0

        lax.fori_loop(0, ngrp, grp, 0)
        plsc.subcore_barrier()
        _acc_dump(acc, out_hbm, c, s, n, rpt, extra_off, extra)

    return k(nbr_i, nbr_j, phiB, DB, zeros128)


def _sc_pass_v(axis, nbr_i, nbr_j, phiC, DCU, DA, AV, zeros128):
    n = phiC.shape[0]
    e2 = nbr_i.shape[0]
    epw = e2 // NW
    eb = 40
    nsup = 10
    nblk = epw // eb
    ngrp = nblk // nsup
    npair = nsup // 2
    rpt, extra_off, extra = _row_split(n)
    has_v = AV is not None

    scratch = [
        pltpu.VMEM_SHARED((n, F), jnp.float32),
        pltpu.VMEM((nsup * eb,), jnp.int32),
        pltpu.VMEM((2, eb), jnp.int32),
        pltpu.VMEM((2 * eb, F), jnp.float32),
        pltpu.VMEM((2 * eb, F), jnp.float32),
        pltpu.SemaphoreType.DMA,
        pltpu.SemaphoreType.DMA,
        pltpu.SemaphoreType.DMA,
    ]
    if has_v:
        scratch += [
            pltpu.VMEM((2 * eb, F), jnp.float32),
            pltpu.VMEM((2 * eb, F), jnp.float32),
        ]

    def body_fn(*refs):
        if has_v:
            (ni_hbm, nj_hbm, pc_hbm, dcu_hbm, da_hbm, av_hbm,
             z_hbm, out_hbm,
             acc, jj_v, ii_v, dc_v, gc_v, semw, sems0, sems1,
             da_v, gav_v) = refs
        else:
            (ni_hbm, nj_hbm, pc_hbm, dcu_hbm,
             z_hbm, out_hbm,
             acc, jj_v, ii_v, dc_v, gc_v, semw, sems0, sems1) = refs
            da_hbm = av_hbm = da_v = gav_v = None
        sems = (sems0, sems1)
        c = lax.axis_index("c")
        s = lax.axis_index("s")
        wid = c * NS + s
        _acc_zero(z_hbm, acc, s, rpt, extra_off, extra)
        plsc.subcore_barrier()

        def fire_wave(off, b, par):
            row = pl.ds(par * eb, eb)
            jslice = jj_v.at[pl.ds(b * eb, eb)]
            pltpu.async_copy(ni_hbm.at[pl.ds(off, eb)], ii_v.at[par], semw)
            pltpu.async_copy(dcu_hbm.at[pl.ds(off, eb), :],
                             dc_v.at[row, :], semw)
            pltpu.async_copy(pc_hbm.at[jslice], gc_v.at[row, :], semw)
            if has_v:
                pltpu.async_copy(da_hbm.at[pl.ds(off, eb), :],
                                 da_v.at[row, :], semw)
                pltpu.async_copy(av_hbm.at[jslice], gav_v.at[row, :], semw)

        def wait_wave(par):
            row = pl.ds(par * eb, eb)
            pltpu.make_async_copy(ni_hbm.at[pl.ds(0, eb)], ii_v.at[par],
                                  semw).wait()
            pltpu.make_async_copy(dcu_hbm.at[pl.ds(0, eb), :],
                                  dc_v.at[row, :], semw).wait()
            pltpu.make_async_copy(pc_hbm.at[pl.ds(0, eb), :],
                                  gc_v.at[row, :], semw).wait()
            if has_v:
                pltpu.make_async_copy(da_hbm.at[pl.ds(0, eb), :],
                                      da_v.at[row, :], semw).wait()
                pltpu.make_async_copy(av_hbm.at[pl.ds(0, eb), :],
                                      gav_v.at[row, :], semw).wait()

        def compute(par):
            base = par * eb

            def edge(ei, _):
                for kk in range(F // LANES):
                    sl = pl.ds(kk * LANES, LANES)
                    val = dc_v[base + ei, sl] * gc_v[base + ei, sl]
                    if has_v:
                        val = val + da_v[base + ei, sl] * gav_v[base + ei, sl]
                    dc_v[base + ei, sl] = val
                return 0

            lax.fori_loop(0, eb, edge, 0)

        def fire_scatter(par):
            row = pl.ds(par * eb, eb)
            pltpu.async_copy(dc_v.at[row, :], acc.at[ii_v.at[par]],
                             sems[par], add=True)

        def wait_scatter(par):
            row = pl.ds(par * eb, eb)
            pltpu.make_async_copy(dc_v.at[row, :], acc.at[ii_v.at[par]],
                                  sems[par]).wait()

        def grp(g, _):
            soff = wid * epw + g * nsup * eb
            pltpu.sync_copy(nj_hbm.at[pl.ds(soff, nsup * eb)], jj_v)
            fire_wave(soff, 0, 0)

            def pair(p, _):
                for par in range(2):
                    b = 2 * p + par
                    wait_wave(par)
                    nxt = b + 1

                    @pl.when(nxt < nsup)
                    def _():
                        fire_wave(soff + nxt * eb, nxt, 1 - par)

                    @pl.when(p > 0)
                    def _():
                        wait_scatter(par)

                    compute(par)
                    fire_scatter(par)
                return 0

            lax.fori_loop(0, npair, pair, 0)
            wait_scatter(0)
            wait_scatter(1)
            return 0

        lax.fori_loop(0, ngrp, grp, 0)
        plsc.subcore_barrier()
        _acc_dump(acc, out_hbm, c, s, n, rpt, extra_off, extra)

    k = pl.kernel(
        body_fn,
        out_type=jax.ShapeDtypeStruct((2 * n, F), jnp.float32),
        mesh=_MESH,
        scratch_types=scratch,
        name=f"sc_pass_v{axis}" + ("" if has_v else "_nov"),
    )
    args = [nbr_i, nbr_j, phiC, DCU]
    if has_v:
        args += [DA, AV]
    args.append(zeros128)
    return k(*args)


def kernel(cg_xyz, CG_nbr_list, cg_s, params):
    n = cg_s.shape[0]
    nbr = jnp.concatenate([CG_nbr_list, CG_nbr_list[:, ::-1]], axis=0)
    nbr_i = nbr[:, 0] + 0
    nbr_j = nbr[:, 1] + 0
    n_orig = CG_nbr_list.shape[0]

    gt_f, lt_f = _directed_flags(CG_nbr_list)
    directed = jnp.logical_and(gt_f[0, 0] > 0, lt_f[0, 0] > 0)
    keep_second = jnp.where(directed, jnp.float32(0.0), jnp.float32(1.0))
    mask = jnp.concatenate([
        jnp.ones((n_orig,), jnp.float32),
        jnp.full((n_orig,), 1.0, jnp.float32) * keep_second,
    ])

    xyzpad = jnp.pad(cg_xyz, ((0, 0), (0, 125)))
    r16 = _xyz_gather(xyzpad, nbr_i, nbr_j)
    r = r16[:, :3]
    dist = jnp.sqrt((r * r + 1e-8).sum(-1))
    unit = r / dist[:, None]
    u16 = jnp.pad(unit, ((0, 0), (0, 13)))
    env = 0.5 * (jnp.cos(jnp.pi * dist / CUTOFF) + 1.0)
    env = jnp.where(dist <= CUTOFF, env, 0.0)
    envm = env * mask
    n_r = jnp.arange(1, NRBF + 1, dtype=jnp.float32)
    rbfenv = (jnp.sin(n_r * (jnp.pi / CUTOFF) * dist[:, None]) / dist[:, None]
              * envm[:, None])
    rbfenv = jnp.pad(rbfenv, ((0, 0), (0, NRBF_PAD - NRBF)))
    envm_col = envm[:, None]

    zeros128 = jnp.zeros((n, F), jnp.float32)

    d_panels = []
    for p in params:
        Wd_pad = jnp.pad(p["Wd"], ((0, NRBF_PAD - NRBF), (0, 0)))
        d_panels.append(_d_panels(rbfenv, envm_col, u16, Wd_pad, p["bd"][None, :]))

    s_cur = cg_s
    p0 = p1 = zeros128
    v_tabs = [None, None, None]
    for li, p in enumerate(params):
        s_cur, phiA, phiB, phiC = _phi_panels(
            s_cur, p0, p1, p["W1"], p["b1"], p["W2"], p["b2"])
        DA, DB, DCx, DCy, DCz = d_panels[li]
        DCU = (DCx, DCy, DCz)

        part_s = _sc_pass_s(nbr_i, nbr_j, phiB, DB, zeros128)
        p0, p1 = part_s[:n], part_s[n:]

        new_tabs = []
        for a in range(3):
            av = None if v_tabs[a] is None else _mul2(phiA, v_tabs[a])
            pv = _sc_pass_v(a, nbr_i, nbr_j, phiC, DCU[a], DA, av, zeros128)
            old = zeros128 if v_tabs[a] is None else v_tabs[a]
            new_tabs.append(_merge3(old, pv[:n], pv[n:]))
        v_tabs = new_tabs

    s_out = _merge3(s_cur, p0, p1)
    v_out = jnp.stack(v_tabs, axis=-1)
    return (s_out, v_out)
